# baseline (speedup 1.0000x reference)
import functools
import math

import jax
import jax.numpy as jnp
from jax import lax
from jax.experimental import pallas as pl
from jax.experimental.pallas import tpu as pltpu
from jax.experimental.pallas import tpu_sc as plsc

_B = 16384
_VOCAB = 1000000
_EMB = 64
_NFREQ = 16
_LATENT = 32

_NC = 2
_NS = 16
_NW = _NC * _NS
_BPW = _B // _NW
_CHUNK = 128
_NCHUNK = _BPW // _CHUNK


def _sc_gather128(table128, idx3):
    mesh = plsc.VectorSubcoreMesh(core_axis_name="c", subcore_axis_name="s")

    @functools.partial(
        pl.kernel,
        mesh=mesh,
        out_type=jax.ShapeDtypeStruct((_B, 128), jnp.float32),
        scratch_types=[
            pltpu.VMEM((_NCHUNK, _CHUNK), jnp.int32),
            pltpu.VMEM((_BPW, 128), jnp.float32),
            pltpu.SemaphoreType.DMA,
        ],
    )
    def k(table_hbm, idx_hbm, out_hbm, idx_v, rows_v, sem):
        wid = lax.axis_index("s") * _NC + lax.axis_index("c")
        base = wid * _BPW
        pltpu.sync_copy(idx_hbm.at[wid], idx_v)
        copies = [
            pltpu.make_async_copy(
                table_hbm.at[idx_v.at[c]],
                rows_v.at[pl.ds(c * _CHUNK, _CHUNK)],
                sem,
            )
            for c in range(_NCHUNK)
        ]
        for cp in copies:
            cp.start()
        for cp in copies:
            cp.wait()
        pltpu.sync_copy(rows_v, out_hbm.at[pl.ds(base, _BPW)])

    return k(table128, idx3)


_TR_BLK = 4096


def _tr_body(in_ref, out_ref):
    eye = jnp.eye(_EMB, dtype=jnp.float32)
    out_ref[:, 0:_EMB] = jax.lax.dot_general(
        in_ref[...], eye, (((0,), (0,)), ((), ())),
        preferred_element_type=jnp.float32)


def _tc_transpose(table_t):
    grid = ((_VOCAB + _TR_BLK - 1) // _TR_BLK,)
    return pl.pallas_call(
        _tr_body,
        grid=grid,
        in_specs=[pl.BlockSpec((_EMB, _TR_BLK), lambda i: (0, i))],
        out_specs=pl.BlockSpec((_TR_BLK, 128), lambda i: (i, 0)),
        out_shape=jax.ShapeDtypeStruct((_VOCAB, 128), jnp.float32),
    )(table_t)


_TC_BLK = 2048


def _tc_body(pos_idx_ref, pos_t_ref, praw_ref, fb_ref, lw_ref, lb_ref,
             out_ref):
    t = pos_t_ref[...]
    proj = (2.0 * math.pi) * (t * fb_ref[...])
    s = jnp.sin(proj)
    c = jnp.cos(proj)
    lat = t * lw_ref[...] + lb_ref[...]
    tenc = jnp.concatenate([s, c, lat], axis=-1)
    prod = praw_ref[:, 0:_EMB] * tenc
    out_ref[:, 0:1] = pos_idx_ref[...]
    out_ref[:, 1:2] = t
    out_ref[:, 2 : 2 + _EMB] = prod


def _tc_dense(pos_idx, pos_t, praw, fourier_B, latent_W, latent_b2):
    grid = (_B // _TC_BLK,)
    return pl.pallas_call(
        _tc_body,
        grid=grid,
        in_specs=[
            pl.BlockSpec((_TC_BLK, 1), lambda i: (i, 0)),
            pl.BlockSpec((_TC_BLK, 1), lambda i: (i, 0)),
            pl.BlockSpec((_TC_BLK, 128), lambda i: (i, 0)),
            pl.BlockSpec((1, _NFREQ), lambda i: (0, 0)),
            pl.BlockSpec((1, _LATENT), lambda i: (0, 0)),
            pl.BlockSpec((1, _LATENT), lambda i: (0, 0)),
        ],
        out_specs=pl.BlockSpec((_TC_BLK, 2 + _EMB), lambda i: (i, 0)),
        out_shape=jax.ShapeDtypeStruct((_B, 2 + _EMB), jnp.float32),
    )(pos_idx, pos_t, praw, fourier_B, latent_W, latent_b2)


def kernel(pos_idx, pos_t, emb_table, fourier_B, latent_W, latent_b):
    table128 = _tc_transpose(emb_table.T)
    idx3 = pos_idx[:, 0].astype(jnp.int32).reshape(_NW, _NCHUNK, _CHUNK)
    praw = _sc_gather128(table128, idx3)
    return _tc_dense(
        pos_idx, pos_t, praw, fourier_B, latent_W,
        latent_b.reshape(1, _LATENT),
    )


# speedup vs baseline: 1.2454x; 1.2454x over previous
import functools
import math

import jax
import jax.numpy as jnp
from jax import lax
from jax.experimental import pallas as pl
from jax.experimental.pallas import tpu as pltpu
from jax.experimental.pallas import tpu_sc as plsc

_B = 16384
_VOCAB = 1000000
_EMB = 64
_NFREQ = 16
_LATENT = 32

_NC = 2
_NS = 16
_NW = _NC * _NS
_BPW = _B // _NW
_CHUNK = 128
_NCHUNK = _BPW // _CHUNK


def _sc_gather128(table128, idx3):
    mesh = plsc.VectorSubcoreMesh(core_axis_name="c", subcore_axis_name="s")

    @functools.partial(
        pl.kernel,
        mesh=mesh,
        out_type=jax.ShapeDtypeStruct((_B, 128), jnp.float32),
        scratch_types=[
            pltpu.VMEM((_NCHUNK, _CHUNK), jnp.int32),
            pltpu.VMEM((_BPW, 128), jnp.float32),
            pltpu.SemaphoreType.DMA,
        ],
    )
    def k(table_hbm, idx_hbm, out_hbm, idx_v, rows_v, sem):
        wid = lax.axis_index("s") * _NC + lax.axis_index("c")
        base = wid * _BPW
        pltpu.sync_copy(idx_hbm.at[wid], idx_v)
        copies = [
            pltpu.make_async_copy(
                table_hbm.at[idx_v.at[c]],
                rows_v.at[pl.ds(c * _CHUNK, _CHUNK)],
                sem,
            )
            for c in range(_NCHUNK)
        ]
        for cp in copies:
            cp.start()
        for cp in copies:
            cp.wait()
        pltpu.sync_copy(rows_v, out_hbm.at[pl.ds(base, _BPW)])

    return k(table128, idx3)


_TR_BLK = 8192


def _tr_body(in_ref, out_ref):
    out_ref[:, 0:_EMB] = in_ref[...].T


def _tc_transpose(table_t):
    grid = ((_VOCAB + _TR_BLK - 1) // _TR_BLK,)
    return pl.pallas_call(
        _tr_body,
        grid=grid,
        in_specs=[pl.BlockSpec((_EMB, _TR_BLK), lambda i: (0, i))],
        out_specs=pl.BlockSpec((_TR_BLK, 128), lambda i: (i, 0)),
        out_shape=jax.ShapeDtypeStruct((_VOCAB, 128), jnp.float32),
    )(table_t)


_TC_BLK = 2048


def _tc_body(pos_idx_ref, pos_t_ref, praw_ref, fb_ref, lw_ref, lb_ref,
             out_ref):
    t = pos_t_ref[...]
    proj = (2.0 * math.pi) * (t * fb_ref[...])
    s = jnp.sin(proj)
    c = jnp.cos(proj)
    lat = t * lw_ref[...] + lb_ref[...]
    tenc = jnp.concatenate([s, c, lat], axis=-1)
    prod = praw_ref[:, 0:_EMB] * tenc
    out_ref[:, 0:1] = pos_idx_ref[...]
    out_ref[:, 1:2] = t
    out_ref[:, 2 : 2 + _EMB] = prod


def _tc_dense(pos_idx, pos_t, praw, fourier_B, latent_W, latent_b2):
    grid = (_B // _TC_BLK,)
    return pl.pallas_call(
        _tc_body,
        grid=grid,
        in_specs=[
            pl.BlockSpec((_TC_BLK, 1), lambda i: (i, 0)),
            pl.BlockSpec((_TC_BLK, 1), lambda i: (i, 0)),
            pl.BlockSpec((_TC_BLK, 128), lambda i: (i, 0)),
            pl.BlockSpec((1, _NFREQ), lambda i: (0, 0)),
            pl.BlockSpec((1, _LATENT), lambda i: (0, 0)),
            pl.BlockSpec((1, _LATENT), lambda i: (0, 0)),
        ],
        out_specs=pl.BlockSpec((_TC_BLK, 2 + _EMB), lambda i: (i, 0)),
        out_shape=jax.ShapeDtypeStruct((_B, 2 + _EMB), jnp.float32),
    )(pos_idx, pos_t, praw, fourier_B, latent_W, latent_b2)


def kernel(pos_idx, pos_t, emb_table, fourier_B, latent_W, latent_b):
    table128 = _tc_transpose(emb_table.T)
    idx3 = pos_idx[:, 0].astype(jnp.int32).reshape(_NW, _NCHUNK, _CHUNK)
    praw = _sc_gather128(table128, idx3)
    return _tc_dense(
        pos_idx, pos_t, praw, fourier_B, latent_W,
        latent_b.reshape(1, _LATENT),
    )


# speedup vs baseline: 1.3132x; 1.0544x over previous
import functools
import math

import jax
import jax.numpy as jnp
from jax import lax
from jax.experimental import pallas as pl
from jax.experimental.pallas import tpu as pltpu
from jax.experimental.pallas import tpu_sc as plsc

_B = 16384
_VOCAB = 1000000
_EMB = 64
_NFREQ = 16
_LATENT = 32

_NC = 2
_NS = 16
_NW = _NC * _NS
_BPW = _B // _NW
_CHUNK = 128
_NCHUNK = _BPW // _CHUNK


def _sc_gather128(table128, idx3):
    mesh = plsc.VectorSubcoreMesh(core_axis_name="c", subcore_axis_name="s")

    @functools.partial(
        pl.kernel,
        mesh=mesh,
        out_type=jax.ShapeDtypeStruct((_B, 128), jnp.float32),
        scratch_types=[
            pltpu.VMEM((_NCHUNK, _CHUNK), jnp.int32),
            pltpu.VMEM((_BPW, 128), jnp.float32),
            pltpu.SemaphoreType.DMA,
        ],
    )
    def k(table_hbm, idx_hbm, out_hbm, idx_v, rows_v, sem):
        wid = lax.axis_index("s") * _NC + lax.axis_index("c")
        base = wid * _BPW
        pltpu.sync_copy(idx_hbm.at[wid], idx_v)
        copies = [
            pltpu.make_async_copy(
                table_hbm.at[idx_v.at[c]],
                rows_v.at[pl.ds(c * _CHUNK, _CHUNK)],
                sem,
            )
            for c in range(_NCHUNK)
        ]
        for cp in copies:
            cp.start()
        for cp in copies:
            cp.wait()
        pltpu.sync_copy(rows_v, out_hbm.at[pl.ds(base, _BPW)])

    return k(table128, idx3)


_TR_BLK = 16384


def _tr_body(in_ref, out_ref):
    out_ref[:, 0:_EMB] = in_ref[...].T


def _tc_transpose(table_t):
    grid = ((_VOCAB + _TR_BLK - 1) // _TR_BLK,)
    return pl.pallas_call(
        _tr_body,
        grid=grid,
        in_specs=[pl.BlockSpec((_EMB, _TR_BLK), lambda i: (0, i))],
        out_specs=pl.BlockSpec((_TR_BLK, 128), lambda i: (i, 0)),
        out_shape=jax.ShapeDtypeStruct((_VOCAB, 128), jnp.float32),
    )(table_t)


_TC_BLK = 2048


def _tc_body(pos_idx_ref, pos_t_ref, praw_ref, fb_ref, lw_ref, lb_ref,
             out_ref):
    t = pos_t_ref[...]
    proj = (2.0 * math.pi) * (t * fb_ref[...])
    s = jnp.sin(proj)
    c = jnp.cos(proj)
    lat = t * lw_ref[...] + lb_ref[...]
    tenc = jnp.concatenate([s, c, lat], axis=-1)
    prod = praw_ref[:, 0:_EMB] * tenc
    out_ref[:, 0:1] = pos_idx_ref[...]
    out_ref[:, 1:2] = t
    out_ref[:, 2 : 2 + _EMB] = prod


def _tc_dense(pos_idx, pos_t, praw, fourier_B, latent_W, latent_b2):
    grid = (_B // _TC_BLK,)
    return pl.pallas_call(
        _tc_body,
        grid=grid,
        in_specs=[
            pl.BlockSpec((_TC_BLK, 1), lambda i: (i, 0)),
            pl.BlockSpec((_TC_BLK, 1), lambda i: (i, 0)),
            pl.BlockSpec((_TC_BLK, 128), lambda i: (i, 0)),
            pl.BlockSpec((1, _NFREQ), lambda i: (0, 0)),
            pl.BlockSpec((1, _LATENT), lambda i: (0, 0)),
            pl.BlockSpec((1, _LATENT), lambda i: (0, 0)),
        ],
        out_specs=pl.BlockSpec((_TC_BLK, 2 + _EMB), lambda i: (i, 0)),
        out_shape=jax.ShapeDtypeStruct((_B, 2 + _EMB), jnp.float32),
    )(pos_idx, pos_t, praw, fourier_B, latent_W, latent_b2)


def kernel(pos_idx, pos_t, emb_table, fourier_B, latent_W, latent_b):
    table128 = _tc_transpose(emb_table.T)
    idx3 = pos_idx[:, 0].astype(jnp.int32).reshape(_NW, _NCHUNK, _CHUNK)
    praw = _sc_gather128(table128, idx3)
    return _tc_dense(
        pos_idx, pos_t, praw, fourier_B, latent_W,
        latent_b.reshape(1, _LATENT),
    )


# speedup vs baseline: 1.3427x; 1.0225x over previous
import functools
import math

import jax
import jax.numpy as jnp
from jax import lax
from jax.experimental import pallas as pl
from jax.experimental.pallas import tpu as pltpu
from jax.experimental.pallas import tpu_sc as plsc

_B = 16384
_VOCAB = 1000000
_EMB = 64
_NFREQ = 16
_LATENT = 32

_NC = 2
_NS = 16
_NW = _NC * _NS
_BPW = _B // _NW
_CHUNK = 128
_NCHUNK = _BPW // _CHUNK


def _sc_gather128(table128, idx3):
    mesh = plsc.VectorSubcoreMesh(core_axis_name="c", subcore_axis_name="s")

    @functools.partial(
        pl.kernel,
        mesh=mesh,
        out_type=jax.ShapeDtypeStruct((_B, 128), jnp.float32),
        scratch_types=[
            pltpu.VMEM((_NCHUNK, _CHUNK), jnp.int32),
            pltpu.VMEM((_BPW, 128), jnp.float32),
            pltpu.SemaphoreType.DMA,
        ],
    )
    def k(table_hbm, idx_hbm, out_hbm, idx_v, rows_v, sem):
        wid = lax.axis_index("s") * _NC + lax.axis_index("c")
        base = wid * _BPW
        pltpu.sync_copy(idx_hbm.at[wid], idx_v)
        copies = [
            pltpu.make_async_copy(
                table_hbm.at[idx_v.at[c]],
                rows_v.at[pl.ds(c * _CHUNK, _CHUNK)],
                sem,
            )
            for c in range(_NCHUNK)
        ]
        for cp in copies:
            cp.start()
        for cp in copies:
            cp.wait()
        pltpu.sync_copy(rows_v, out_hbm.at[pl.ds(base, _BPW)])

    return k(table128, idx3)


_TR_BLK = 32768


def _tr_body(in_ref, out_ref):
    out_ref[:, 0:_EMB] = in_ref[...].T


def _tc_transpose(table_t):
    grid = ((_VOCAB + _TR_BLK - 1) // _TR_BLK,)
    return pl.pallas_call(
        _tr_body,
        grid=grid,
        in_specs=[pl.BlockSpec((_EMB, _TR_BLK), lambda i: (0, i))],
        out_specs=pl.BlockSpec((_TR_BLK, 128), lambda i: (i, 0)),
        out_shape=jax.ShapeDtypeStruct((_VOCAB, 128), jnp.float32),
    )(table_t)


_TC_BLK = 2048


def _tc_body(pos_idx_ref, pos_t_ref, praw_ref, fb_ref, lw_ref, lb_ref,
             out_ref):
    t = pos_t_ref[...]
    proj = (2.0 * math.pi) * (t * fb_ref[...])
    s = jnp.sin(proj)
    c = jnp.cos(proj)
    lat = t * lw_ref[...] + lb_ref[...]
    tenc = jnp.concatenate([s, c, lat], axis=-1)
    prod = praw_ref[:, 0:_EMB] * tenc
    out_ref[:, 0:1] = pos_idx_ref[...]
    out_ref[:, 1:2] = t
    out_ref[:, 2 : 2 + _EMB] = prod


def _tc_dense(pos_idx, pos_t, praw, fourier_B, latent_W, latent_b2):
    grid = (_B // _TC_BLK,)
    return pl.pallas_call(
        _tc_body,
        grid=grid,
        in_specs=[
            pl.BlockSpec((_TC_BLK, 1), lambda i: (i, 0)),
            pl.BlockSpec((_TC_BLK, 1), lambda i: (i, 0)),
            pl.BlockSpec((_TC_BLK, 128), lambda i: (i, 0)),
            pl.BlockSpec((1, _NFREQ), lambda i: (0, 0)),
            pl.BlockSpec((1, _LATENT), lambda i: (0, 0)),
            pl.BlockSpec((1, _LATENT), lambda i: (0, 0)),
        ],
        out_specs=pl.BlockSpec((_TC_BLK, 2 + _EMB), lambda i: (i, 0)),
        out_shape=jax.ShapeDtypeStruct((_B, 2 + _EMB), jnp.float32),
    )(pos_idx, pos_t, praw, fourier_B, latent_W, latent_b2)


def kernel(pos_idx, pos_t, emb_table, fourier_B, latent_W, latent_b):
    table128 = _tc_transpose(emb_table.T)
    idx3 = pos_idx[:, 0].astype(jnp.int32).reshape(_NW, _NCHUNK, _CHUNK)
    praw = _sc_gather128(table128, idx3)
    return _tc_dense(
        pos_idx, pos_t, praw, fourier_B, latent_W,
        latent_b.reshape(1, _LATENT),
    )
